# baseline (speedup 1.0000x reference)
import jax
import jax.numpy as jnp
from jax.experimental import pallas as pl

N = 10000
C = 128
G = 64
EPS = 1e-4


def _enc(p):
    return (p[:, 0] * G + p[:, 1]) * G + p[:, 2]


def _nbrs(pos_int):
    keys = _enc(pos_int)
    order = jnp.argsort(keys)
    skeys = keys[order]
    offs = jnp.array([[i, j, k] for i in (-1, 0, 1) for j in (-1, 0, 1)
                      for k in (-1, 0, 1)], dtype=jnp.int32)

    def per_off(off):
        q = pos_int + off
        valid = jnp.all((q >= 0) & (q < G), axis=1)
        qk = _enc(q)
        loc = jnp.clip(jnp.searchsorted(skeys, qk), 0, N - 1)
        found = (skeys[loc] == qk) & valid
        return jnp.where(found, order[loc], -1)

    return jax.vmap(per_off)(offs).T


def _bn(x, w, b):
    mean = jnp.mean(x, axis=0)
    var = jnp.mean((x - mean) ** 2, axis=0)
    return (x - mean) * jax.lax.rsqrt(var + EPS) * w + b


def _conv(feat, nbr, W):
    mask = (nbr >= 0)[:, :, None]
    g = jnp.where(mask, feat[jnp.clip(nbr, 0)], 0.0)
    return jnp.einsum('nkc,kcd->nd', g, W)


def _add_kernel(a_ref, b_ref, o_ref):
    o_ref[...] = a_ref[...] + b_ref[...]


def kernel(feat_list, pos_list, W1, W2, bn1_w, bn1_b, bn2_w, bn2_b):
    pos_int = pos_list.astype(jnp.int32)
    nbr = _nbrs(pos_int)
    x = _bn(feat_list, bn1_w, bn1_b)
    x = jax.nn.relu(x)
    x = _conv(x, nbr, W1)
    x = _bn(x, bn2_w, bn2_b)
    x = jax.nn.relu(x)
    out2 = _conv(x, nbr, W2)
    return pl.pallas_call(
        _add_kernel,
        out_shape=jax.ShapeDtypeStruct((N, C), jnp.float32),
    )(feat_list, out2)


# speedup vs baseline: 1.1396x; 1.1396x over previous
import functools

import jax
import jax.numpy as jnp
from jax import lax
from jax.experimental import pallas as pl
from jax.experimental.pallas import tpu as pltpu
from jax.experimental.pallas import tpu_sc as plsc

N = 10000
C = 128
G = 64
EPS = 1e-4

NC = 2
NS = 16
NW = NC * NS
CH = 320
NPAD = NW * CH
SENT = NPAD
XROWS = NPAD + 16
TBL = 294912
QCHUNKS = 68
RB = 320
NRB = NPAD // RB
K27 = 27

_mesh = plsc.VectorSubcoreMesh(
    core_axis_name="c", subcore_axis_name="s", num_cores=NC, num_subcores=NS)


def _nbr_body(posx, posy, posz, fidx_out,
              table, px_v, py_v, pz_v, key_v, val_v, qbuf, vbuf, tv, fo,
              neg_v, sem):
    c = lax.axis_index("c")
    s = lax.axis_index("s")
    wid = c * NS + s
    base = wid * CH

    def memset_loop(i, _):
        neg_v[pl.ds(i * 16, 16)] = jnp.full((16,), -1, jnp.int32)
        return 0
    lax.fori_loop(0, neg_v.shape[0] // 16, memset_loop, 0)
    seg = TBL // NS
    nfill = seg // neg_v.shape[0]
    for r in range(nfill):
        pltpu.sync_copy(neg_v, table.at[pl.ds(s * seg + r * neg_v.shape[0],
                                              neg_v.shape[0])])
    plsc.subcore_barrier()

    for half in range(2):
        cbase = (half * NS + s) * CH
        pltpu.sync_copy(posx.at[pl.ds(cbase, CH)], px_v)
        pltpu.sync_copy(posy.at[pl.ds(cbase, CH)], py_v)
        pltpu.sync_copy(posz.at[pl.ds(cbase, CH)], pz_v)

        def key_loop(i, _):
            x = px_v[pl.ds(i * 16, 16)]
            y = py_v[pl.ds(i * 16, 16)]
            z = pz_v[pl.ds(i * 16, 16)]
            k16 = (x * G + y) * G + z
            lane = lax.broadcasted_iota(jnp.int32, (16,), 0)
            r = i // 4
            col = (i % 4) * 16
            key_v[r, pl.ds(col, 16)] = k16
            val_v[r, pl.ds(col, 16)] = cbase + i * 16 + lane
            return 0
        lax.fori_loop(0, CH // 16, key_loop, 0)
        for r in range(CH // 64):
            pltpu.sync_copy(val_v.at[r], table.at[key_v.at[r]])
    plsc.subcore_barrier()

    for t in range(4):
        qbuf[QCHUNKS - 1, pl.ds(64 + t * 16, 16)] = jnp.zeros((16,),
                                                              jnp.int32)
    pltpu.sync_copy(posx.at[pl.ds(base, CH)], px_v)
    pltpu.sync_copy(posy.at[pl.ds(base, CH)], py_v)
    pltpu.sync_copy(posz.at[pl.ds(base, CH)], pz_v)

    def q_loop(koff, _):
        di = koff // 9 - 1
        dj = (koff // 3) % 3 - 1
        dk = koff % 3 - 1

        def q_inner(i, _):
            x = px_v[pl.ds(i * 16, 16)] + di
            y = py_v[pl.ds(i * 16, 16)] + dj
            z = pz_v[pl.ds(i * 16, 16)] + dk
            lane = lax.broadcasted_iota(jnp.int32, (16,), 0)
            nidx = base + i * 16 + lane
            inb = ((x >= 0) & (x < G) & (y >= 0) & (y < G)
                   & (z >= 0) & (z < G) & (nidx < N))
            qk = jnp.clip((x * G + y) * G + z, 0, TBL - 1)
            p = koff * CH + i * 16
            qbuf[p // 128, pl.ds(p % 128, 16)] = qk
            vbuf[pl.ds(p, 16)] = jnp.where(inb, 1, 0).astype(jnp.int32)
            return 0
        lax.fori_loop(0, CH // 16, q_inner, 0)
        return 0
    lax.fori_loop(0, K27, q_loop, 0)

    def g_loop(j, _):
        pltpu.sync_copy(table.at[qbuf.at[j]], tv.at[j])
        return 0
    lax.fori_loop(0, QCHUNKS, g_loop, 0)

    def f_loop(i, _):
        p = i * 16
        t = tv[p // 128, pl.ds(p % 128, 16)]
        v = vbuf[pl.ds(p, 16)]
        found = (t >= 0) & (v > 0)
        fo[pl.ds(p, 16)] = jnp.where(found, t, SENT).astype(jnp.int32)
        return 0
    lax.fori_loop(0, (K27 * CH) // 16, f_loop, 0)

    def w_loop(koff, _):
        pltpu.sync_copy(fo.at[pl.ds(koff * CH, CH)],
                        fidx_out.at[pl.ds(koff * NPAD + base, CH)])
        return 0
    lax.fori_loop(0, K27, w_loop, 0)


_nbr_kernel = pl.kernel(
    _nbr_body,
    out_type=jax.ShapeDtypeStruct((K27 * NPAD,), jnp.int32),
    mesh=_mesh,
    scratch_types=[
        pltpu.VMEM_SHARED((TBL,), jnp.int32),
        pltpu.VMEM((CH,), jnp.int32),
        pltpu.VMEM((CH,), jnp.int32),
        pltpu.VMEM((CH,), jnp.int32),
        pltpu.VMEM((CH // 64, 64), jnp.int32),
        pltpu.VMEM((CH // 64, 64), jnp.int32),
        pltpu.VMEM((QCHUNKS, 128), jnp.int32),
        pltpu.VMEM((QCHUNKS * 128,), jnp.int32),
        pltpu.VMEM((QCHUNKS, 128), jnp.int32),
        pltpu.VMEM((K27 * CH,), jnp.int32),
        pltpu.VMEM((2048,), jnp.int32),
        pltpu.SemaphoreType.DMA,
    ],
)


def _gather_body(xpad, fidx, g_out, idx_v, rows_v, sem):
    c = lax.axis_index("c")
    s = lax.axis_index("s")
    wid = c * NS + s
    base = wid * CH

    def k_loop(koff, _):
        pltpu.sync_copy(fidx.at[pl.ds(koff * NPAD + base, CH)], idx_v)
        for off, ln in ((0, 128), (128, 128), (256, 64)):
            pltpu.async_copy(xpad.at[idx_v.at[pl.ds(off, ln)]],
                             rows_v.at[pl.ds(off, ln)], sem).wait()
        pltpu.sync_copy(rows_v, g_out.at[koff, pl.ds(base, CH)])
        return 0
    lax.fori_loop(0, K27, k_loop, 0)


_gather_kernel = pl.kernel(
    _gather_body,
    out_type=jax.ShapeDtypeStruct((K27, NPAD, C), jnp.float32),
    mesh=_mesh,
    scratch_types=[
        pltpu.VMEM((CH,), jnp.int32),
        pltpu.VMEM((CH, C), jnp.float32),
        pltpu.SemaphoreType.DMA,
    ],
)


def _bn_relu_body(x_ref, w_ref, b_ref, o_ref):
    x = x_ref[...]
    ri = lax.broadcasted_iota(jnp.int32, (NPAD, 1), 0)
    msk = ri < N
    mean = jnp.sum(x, axis=0, keepdims=True) * (1.0 / N)
    d = x - mean
    dm = jnp.where(msk, d, 0.0)
    var = jnp.sum(dm * dm, axis=0, keepdims=True) * (1.0 / N)
    y = d * lax.rsqrt(var + EPS) * w_ref[...] + b_ref[...]
    y = jnp.maximum(y, 0.0)
    o_ref[pl.ds(0, NPAD), :] = jnp.where(msk, y, 0.0)
    o_ref[pl.ds(NPAD, XROWS - NPAD), :] = jnp.zeros((XROWS - NPAD, C),
                                                    jnp.float32)


def _bn_relu(xpad, w, b):
    return pl.pallas_call(
        _bn_relu_body,
        out_shape=jax.ShapeDtypeStruct((XROWS, C), jnp.float32),
    )(xpad, w.reshape(1, C), b.reshape(1, C))


def _mm_body(g_ref, w_ref, o_ref):
    k = pl.program_id(1)
    part = jnp.dot(g_ref[0], w_ref[0], preferred_element_type=jnp.float32)

    @pl.when(k == 0)
    def _():
        o_ref[...] = part

    @pl.when(k > 0)
    def _():
        o_ref[...] += part


def _mm_res_body(g_ref, w_ref, f_ref, o_ref):
    k = pl.program_id(1)
    part = jnp.dot(g_ref[0], w_ref[0], preferred_element_type=jnp.float32)

    @pl.when(k == 0)
    def _():
        o_ref[...] = f_ref[...] + part

    @pl.when(k > 0)
    def _():
        o_ref[...] += part


def _conv_mm(g, w):
    return pl.pallas_call(
        _mm_body,
        grid=(NRB, K27),
        in_specs=[
            pl.BlockSpec((1, RB, C), lambda j, k: (k, j, 0)),
            pl.BlockSpec((1, C, C), lambda j, k: (k, 0, 0)),
        ],
        out_specs=pl.BlockSpec((RB, C), lambda j, k: (j, 0)),
        out_shape=jax.ShapeDtypeStruct((NPAD, C), jnp.float32),
    )(g, w)


def _conv_mm_res(g, w, fpad):
    return pl.pallas_call(
        _mm_res_body,
        grid=(NRB, K27),
        in_specs=[
            pl.BlockSpec((1, RB, C), lambda j, k: (k, j, 0)),
            pl.BlockSpec((1, C, C), lambda j, k: (k, 0, 0)),
            pl.BlockSpec((RB, C), lambda j, k: (j, 0)),
        ],
        out_specs=pl.BlockSpec((RB, C), lambda j, k: (j, 0)),
        out_shape=jax.ShapeDtypeStruct((NPAD, C), jnp.float32),
    )(g, w, fpad)


def kernel(feat_list, pos_list, W1, W2, bn1_w, bn1_b, bn2_w, bn2_b):
    pos_i = pos_list.astype(jnp.int32)
    j = jnp.arange(NPAD - N, dtype=jnp.int32)
    pad_pos = jnp.stack([jnp.full_like(j, G), j // G, j % G], axis=1)
    posT = jnp.concatenate([pos_i, pad_pos], axis=0).T
    posx, posy, posz = posT[0], posT[1], posT[2]

    fidx = _nbr_kernel(posx, posy, posz)

    featpad = jnp.pad(feat_list, ((0, NPAD - N), (0, 0)))
    x1 = _bn_relu(featpad, bn1_w, bn1_b)
    g1 = _gather_kernel(x1, fidx)
    h = _conv_mm(g1, W1)
    x2 = _bn_relu(h, bn2_w, bn2_b)
    g2 = _gather_kernel(x2, fidx)
    outpad = _conv_mm_res(g2, W2, featpad)
    return outpad[:N]


# speedup vs baseline: 1.1406x; 1.0009x over previous
import functools

import jax
import jax.numpy as jnp
from jax import lax
from jax.experimental import pallas as pl
from jax.experimental.pallas import tpu as pltpu
from jax.experimental.pallas import tpu_sc as plsc

N = 10000
C = 128
G = 64
EPS = 1e-4

NC = 2
NS = 16
NW = NC * NS
CH = 320
NPAD = NW * CH
SENT = NPAD
XROWS = NPAD + 16
TBL = 294912
QCHUNKS = 68
RB = 320
NRB = NPAD // RB
K27 = 27

_mesh = plsc.VectorSubcoreMesh(
    core_axis_name="c", subcore_axis_name="s", num_cores=NC, num_subcores=NS)


def _nbr_body(posx, posy, posz, fidx_out,
              table, px_v, py_v, pz_v, key_v, val_v, qbuf, vbuf, tv, fo,
              neg_v, sem):
    c = lax.axis_index("c")
    s = lax.axis_index("s")
    wid = c * NS + s
    base = wid * CH

    def memset_loop(i, _):
        neg_v[pl.ds(i * 16, 16)] = jnp.full((16,), -1, jnp.int32)
        return 0
    lax.fori_loop(0, neg_v.shape[0] // 16, memset_loop, 0)
    seg = TBL // NS
    nfill = seg // neg_v.shape[0]
    for r in range(nfill):
        pltpu.sync_copy(neg_v, table.at[pl.ds(s * seg + r * neg_v.shape[0],
                                              neg_v.shape[0])])
    plsc.subcore_barrier()

    for half in range(2):
        cbase = (half * NS + s) * CH
        pltpu.sync_copy(posx.at[pl.ds(cbase, CH)], px_v)
        pltpu.sync_copy(posy.at[pl.ds(cbase, CH)], py_v)
        pltpu.sync_copy(posz.at[pl.ds(cbase, CH)], pz_v)

        def key_loop(i, _):
            x = px_v[pl.ds(i * 16, 16)]
            y = py_v[pl.ds(i * 16, 16)]
            z = pz_v[pl.ds(i * 16, 16)]
            k16 = (x * G + y) * G + z
            lane = lax.broadcasted_iota(jnp.int32, (16,), 0)
            r = i // 4
            col = (i % 4) * 16
            key_v[r, pl.ds(col, 16)] = k16
            val_v[r, pl.ds(col, 16)] = cbase + i * 16 + lane
            return 0
        lax.fori_loop(0, CH // 16, key_loop, 0)
        for r in range(CH // 64):
            pltpu.sync_copy(val_v.at[r], table.at[key_v.at[r]])
    plsc.subcore_barrier()

    for t in range(4):
        qbuf[QCHUNKS - 1, pl.ds(64 + t * 16, 16)] = jnp.zeros((16,),
                                                              jnp.int32)
    pltpu.sync_copy(posx.at[pl.ds(base, CH)], px_v)
    pltpu.sync_copy(posy.at[pl.ds(base, CH)], py_v)
    pltpu.sync_copy(posz.at[pl.ds(base, CH)], pz_v)

    def q_loop(koff, _):
        di = koff // 9 - 1
        dj = (koff // 3) % 3 - 1
        dk = koff % 3 - 1

        def q_inner(i, _):
            x = px_v[pl.ds(i * 16, 16)] + di
            y = py_v[pl.ds(i * 16, 16)] + dj
            z = pz_v[pl.ds(i * 16, 16)] + dk
            lane = lax.broadcasted_iota(jnp.int32, (16,), 0)
            nidx = base + i * 16 + lane
            inb = ((x >= 0) & (x < G) & (y >= 0) & (y < G)
                   & (z >= 0) & (z < G) & (nidx < N))
            qk = jnp.clip((x * G + y) * G + z, 0, TBL - 1)
            p = koff * CH + i * 16
            qbuf[p // 128, pl.ds(p % 128, 16)] = qk
            vbuf[pl.ds(p, 16)] = jnp.where(inb, 1, 0).astype(jnp.int32)
            return 0
        lax.fori_loop(0, CH // 16, q_inner, 0)
        return 0
    lax.fori_loop(0, K27, q_loop, 0)

    def g_loop(j, _):
        pltpu.sync_copy(table.at[qbuf.at[j]], tv.at[j])
        return 0
    lax.fori_loop(0, QCHUNKS, g_loop, 0)

    def f_loop(i, _):
        p = i * 16
        t = tv[p // 128, pl.ds(p % 128, 16)]
        v = vbuf[pl.ds(p, 16)]
        found = (t >= 0) & (v > 0)
        fo[pl.ds(p, 16)] = jnp.where(found, t, SENT).astype(jnp.int32)
        return 0
    lax.fori_loop(0, (K27 * CH) // 16, f_loop, 0)

    pltpu.sync_copy(fo, fidx_out.at[pl.ds(wid * K27 * CH, K27 * CH)])


_nbr_kernel = pl.kernel(
    _nbr_body,
    out_type=jax.ShapeDtypeStruct((NW * K27 * CH,), jnp.int32),
    mesh=_mesh,
    scratch_types=[
        pltpu.VMEM_SHARED((TBL,), jnp.int32),
        pltpu.VMEM((CH,), jnp.int32),
        pltpu.VMEM((CH,), jnp.int32),
        pltpu.VMEM((CH,), jnp.int32),
        pltpu.VMEM((CH // 64, 64), jnp.int32),
        pltpu.VMEM((CH // 64, 64), jnp.int32),
        pltpu.VMEM((QCHUNKS, 128), jnp.int32),
        pltpu.VMEM((QCHUNKS * 128,), jnp.int32),
        pltpu.VMEM((QCHUNKS, 128), jnp.int32),
        pltpu.VMEM((K27 * CH,), jnp.int32),
        pltpu.VMEM((2048,), jnp.int32),
        pltpu.SemaphoreType.DMA,
    ],
)


GCHUNK = 432
NGCH = (K27 * CH) // GCHUNK


def _gather_body(xpad, fidx, g_out, idx_v, rows_a, rows_b, sem_g, sem_w):
    c = lax.axis_index("c")
    s = lax.axis_index("s")
    wid = c * NS + s
    gbase = wid * K27 * CH

    pltpu.sync_copy(fidx.at[pl.ds(gbase, K27 * CH)], idx_v)
    bufs = (rows_a, rows_b)
    gd = [None] * NGCH
    wd = [None] * NGCH
    for ci in range(NGCH):
        if ci >= 2:
            wd[ci - 2].wait()
        gd[ci] = pltpu.async_copy(
            xpad.at[idx_v.at[pl.ds(ci * GCHUNK, GCHUNK)]],
            bufs[ci % 2], sem_g)
        if ci >= 1:
            gd[ci - 1].wait()
            wd[ci - 1] = pltpu.async_copy(
                bufs[(ci - 1) % 2],
                g_out.at[wid, pl.ds((ci - 1) * GCHUNK, GCHUNK)], sem_w)
    gd[NGCH - 1].wait()
    wd[NGCH - 1] = pltpu.async_copy(
        bufs[(NGCH - 1) % 2],
        g_out.at[wid, pl.ds((NGCH - 1) * GCHUNK, GCHUNK)], sem_w)
    wd[NGCH - 2].wait()
    wd[NGCH - 1].wait()


_gather_kernel = pl.kernel(
    _gather_body,
    out_type=jax.ShapeDtypeStruct((NW, K27 * CH, C), jnp.float32),
    mesh=_mesh,
    scratch_types=[
        pltpu.VMEM((K27 * CH,), jnp.int32),
        pltpu.VMEM((GCHUNK, C), jnp.float32),
        pltpu.VMEM((GCHUNK, C), jnp.float32),
        pltpu.SemaphoreType.DMA,
        pltpu.SemaphoreType.DMA,
    ],
)


def _bn_relu_body(x_ref, w_ref, b_ref, o_ref):
    x = x_ref[...]
    ri = lax.broadcasted_iota(jnp.int32, (NPAD, 1), 0)
    msk = ri < N
    mean = jnp.sum(x, axis=0, keepdims=True) * (1.0 / N)
    d = x - mean
    dm = jnp.where(msk, d, 0.0)
    var = jnp.sum(dm * dm, axis=0, keepdims=True) * (1.0 / N)
    y = d * lax.rsqrt(var + EPS) * w_ref[...] + b_ref[...]
    y = jnp.maximum(y, 0.0)
    o_ref[pl.ds(0, NPAD), :] = jnp.where(msk, y, 0.0)
    o_ref[pl.ds(NPAD, XROWS - NPAD), :] = jnp.zeros((XROWS - NPAD, C),
                                                    jnp.float32)


def _bn_relu(xpad, w, b):
    return pl.pallas_call(
        _bn_relu_body,
        out_shape=jax.ShapeDtypeStruct((XROWS, C), jnp.float32),
    )(xpad, w.reshape(1, C), b.reshape(1, C))


def _mm_body(g_ref, w_ref, o_ref):
    k = pl.program_id(1)
    part = jnp.dot(g_ref[0], w_ref[0], preferred_element_type=jnp.float32)

    @pl.when(k == 0)
    def _():
        o_ref[...] = part

    @pl.when(k > 0)
    def _():
        o_ref[...] += part


def _mm_res_body(g_ref, w_ref, f_ref, o_ref):
    k = pl.program_id(1)
    part = jnp.dot(g_ref[0], w_ref[0], preferred_element_type=jnp.float32)

    @pl.when(k == 0)
    def _():
        o_ref[...] = f_ref[...] + part

    @pl.when(k > 0)
    def _():
        o_ref[...] += part


def _conv_mm(g, w):
    return pl.pallas_call(
        _mm_body,
        grid=(NRB, K27),
        in_specs=[
            pl.BlockSpec((1, RB, C), lambda j, k: (j, k, 0)),
            pl.BlockSpec((1, C, C), lambda j, k: (k, 0, 0)),
        ],
        out_specs=pl.BlockSpec((RB, C), lambda j, k: (j, 0)),
        out_shape=jax.ShapeDtypeStruct((NPAD, C), jnp.float32),
    )(g, w)


def _conv_mm_res(g, w, fpad):
    return pl.pallas_call(
        _mm_res_body,
        grid=(NRB, K27),
        in_specs=[
            pl.BlockSpec((1, RB, C), lambda j, k: (j, k, 0)),
            pl.BlockSpec((1, C, C), lambda j, k: (k, 0, 0)),
            pl.BlockSpec((RB, C), lambda j, k: (j, 0)),
        ],
        out_specs=pl.BlockSpec((RB, C), lambda j, k: (j, 0)),
        out_shape=jax.ShapeDtypeStruct((NPAD, C), jnp.float32),
    )(g, w, fpad)


def kernel(feat_list, pos_list, W1, W2, bn1_w, bn1_b, bn2_w, bn2_b):
    pos_i = pos_list.astype(jnp.int32)
    j = jnp.arange(NPAD - N, dtype=jnp.int32)
    pad_pos = jnp.stack([jnp.full_like(j, G), j // G, j % G], axis=1)
    posT = jnp.concatenate([pos_i, pad_pos], axis=0).T
    posx, posy, posz = posT[0], posT[1], posT[2]

    fidx = _nbr_kernel(posx, posy, posz)

    featpad = jnp.pad(feat_list, ((0, NPAD - N), (0, 0)))
    x1 = _bn_relu(featpad, bn1_w, bn1_b)
    g1 = _gather_kernel(x1, fidx)
    h = _conv_mm(g1, W1)
    x2 = _bn_relu(h, bn2_w, bn2_b)
    g2 = _gather_kernel(x2, fidx)
    outpad = _conv_mm_res(g2, W2, featpad)
    return outpad[:N]


# speedup vs baseline: 18.3622x; 16.0985x over previous
import functools

import jax
import jax.numpy as jnp
from jax import lax
from jax.experimental import pallas as pl
from jax.experimental.pallas import tpu as pltpu
from jax.experimental.pallas import tpu_sc as plsc

N = 10000
C = 128
G = 64
EPS = 1e-4

NC = 2
NS = 16
NW = NC * NS
CH = 320
NPAD = NW * CH
SENT = NPAD
XROWS = 10368
TBL = 294912
QCHUNKS = 68
RB = 320
NRB = NPAD // RB
K27 = 27

_mesh = plsc.VectorSubcoreMesh(
    core_axis_name="c", subcore_axis_name="s", num_cores=NC, num_subcores=NS)


def _nbr_body(posx, posy, posz, fidx_out,
              table, px_v, py_v, pz_v, key_v, val_v, qbuf, vbuf, tv, fo,
              neg_v, sem):
    c = lax.axis_index("c")
    s = lax.axis_index("s")
    wid = c * NS + s
    base = wid * CH

    def memset_loop(i, _):
        neg_v[pl.ds(i * 16, 16)] = jnp.full((16,), -1, jnp.int32)
        return 0
    lax.fori_loop(0, neg_v.shape[0] // 16, memset_loop, 0)
    seg = TBL // NS
    nfill = seg // neg_v.shape[0]
    for r in range(nfill):
        pltpu.sync_copy(neg_v, table.at[pl.ds(s * seg + r * neg_v.shape[0],
                                              neg_v.shape[0])])
    plsc.subcore_barrier()

    for half in range(2):
        cbase = (half * NS + s) * CH
        pltpu.sync_copy(posx.at[pl.ds(cbase, CH)], px_v)
        pltpu.sync_copy(posy.at[pl.ds(cbase, CH)], py_v)
        pltpu.sync_copy(posz.at[pl.ds(cbase, CH)], pz_v)

        def key_loop(i, _):
            x = px_v[pl.ds(i * 16, 16)]
            y = py_v[pl.ds(i * 16, 16)]
            z = pz_v[pl.ds(i * 16, 16)]
            k16 = (x * G + y) * G + z
            lane = lax.broadcasted_iota(jnp.int32, (16,), 0)
            r = i // 4
            col = (i % 4) * 16
            key_v[r, pl.ds(col, 16)] = k16
            val_v[r, pl.ds(col, 16)] = cbase + i * 16 + lane
            return 0
        lax.fori_loop(0, CH // 16, key_loop, 0)
        for r in range(CH // 64):
            pltpu.sync_copy(val_v.at[r], table.at[key_v.at[r]])
    plsc.subcore_barrier()

    for t in range(4):
        qbuf[QCHUNKS - 1, pl.ds(64 + t * 16, 16)] = jnp.zeros((16,),
                                                              jnp.int32)
    pltpu.sync_copy(posx.at[pl.ds(base, CH)], px_v)
    pltpu.sync_copy(posy.at[pl.ds(base, CH)], py_v)
    pltpu.sync_copy(posz.at[pl.ds(base, CH)], pz_v)

    def q_loop(koff, _):
        di = koff // 9 - 1
        dj = (koff // 3) % 3 - 1
        dk = koff % 3 - 1

        def q_inner(i, _):
            x = px_v[pl.ds(i * 16, 16)] + di
            y = py_v[pl.ds(i * 16, 16)] + dj
            z = pz_v[pl.ds(i * 16, 16)] + dk
            lane = lax.broadcasted_iota(jnp.int32, (16,), 0)
            nidx = base + i * 16 + lane
            inb = ((x >= 0) & (x < G) & (y >= 0) & (y < G)
                   & (z >= 0) & (z < G) & (nidx < N))
            qk = jnp.clip((x * G + y) * G + z, 0, TBL - 1)
            p = koff * CH + i * 16
            qbuf[p // 128, pl.ds(p % 128, 16)] = qk
            vbuf[pl.ds(p, 16)] = jnp.where(inb, 1, 0).astype(jnp.int32)
            return 0
        lax.fori_loop(0, CH // 16, q_inner, 0)
        return 0
    lax.fori_loop(0, K27, q_loop, 0)

    def g_loop(j, _):
        pltpu.sync_copy(table.at[qbuf.at[j]], tv.at[j])
        return 0
    lax.fori_loop(0, QCHUNKS, g_loop, 0)

    def f_loop(i, _):
        p = i * 16
        t = tv[p // 128, pl.ds(p % 128, 16)]
        v = vbuf[pl.ds(p, 16)]
        found = (t >= 0) & (v > 0)
        fo[pl.ds(p, 16)] = jnp.where(found, t, SENT).astype(jnp.int32)
        return 0
    lax.fori_loop(0, (K27 * CH) // 16, f_loop, 0)

    pltpu.sync_copy(fo, fidx_out.at[pl.ds(wid * K27 * CH, K27 * CH)])


_nbr_kernel = pl.kernel(
    _nbr_body,
    out_type=jax.ShapeDtypeStruct((NW * K27 * CH,), jnp.int32),
    mesh=_mesh,
    scratch_types=[
        pltpu.VMEM_SHARED((TBL,), jnp.int32),
        pltpu.VMEM((CH,), jnp.int32),
        pltpu.VMEM((CH,), jnp.int32),
        pltpu.VMEM((CH,), jnp.int32),
        pltpu.VMEM((CH // 64, 64), jnp.int32),
        pltpu.VMEM((CH // 64, 64), jnp.int32),
        pltpu.VMEM((QCHUNKS, 128), jnp.int32),
        pltpu.VMEM((QCHUNKS * 128,), jnp.int32),
        pltpu.VMEM((QCHUNKS, 128), jnp.int32),
        pltpu.VMEM((K27 * CH,), jnp.int32),
        pltpu.VMEM((2048,), jnp.int32),
        pltpu.SemaphoreType.DMA,
    ],
)


GCHUNK = 120
NGCH = (K27 * CH) // GCHUNK


def _gather_body(xpad, fidx, g_out, xsh, idx_v, rows_a, rows_b, sem_g, sem_w):
    c = lax.axis_index("c")
    s = lax.axis_index("s")
    wid = c * NS + s
    gbase = wid * K27 * CH

    cps = XROWS // NS
    off = 0
    for ln in (120, 120, 120, 120, 120, 48):
        r0 = s * cps + off
        pltpu.sync_copy(xpad.at[pl.ds(r0, ln)], rows_a.at[pl.ds(0, ln)])
        pltpu.sync_copy(rows_a.at[pl.ds(0, ln)], xsh.at[pl.ds(r0, ln)])
        off += ln
    plsc.subcore_barrier()

    pltpu.sync_copy(fidx.at[pl.ds(gbase, K27 * CH)], idx_v)
    bufs = (rows_a, rows_b)
    gd = [None] * NGCH
    wd = [None] * NGCH
    for ci in range(NGCH):
        if ci >= 2:
            wd[ci - 2].wait()
        gd[ci] = pltpu.async_copy(
            xsh.at[idx_v.at[pl.ds(ci * GCHUNK, GCHUNK)]],
            bufs[ci % 2], sem_g)
        if ci >= 1:
            gd[ci - 1].wait()
            wd[ci - 1] = pltpu.async_copy(
                bufs[(ci - 1) % 2],
                g_out.at[wid, pl.ds((ci - 1) * GCHUNK, GCHUNK)], sem_w)
    gd[NGCH - 1].wait()
    wd[NGCH - 1] = pltpu.async_copy(
        bufs[(NGCH - 1) % 2],
        g_out.at[wid, pl.ds((NGCH - 1) * GCHUNK, GCHUNK)], sem_w)
    wd[NGCH - 2].wait()
    wd[NGCH - 1].wait()


_gather_kernel = pl.kernel(
    _gather_body,
    out_type=jax.ShapeDtypeStruct((NW, K27 * CH, C), jnp.float32),
    mesh=_mesh,
    scratch_types=[
        pltpu.VMEM_SHARED((XROWS, C), jnp.float32),
        pltpu.VMEM((K27 * CH,), jnp.int32),
        pltpu.VMEM((GCHUNK, C), jnp.float32),
        pltpu.VMEM((GCHUNK, C), jnp.float32),
        pltpu.SemaphoreType.DMA,
        pltpu.SemaphoreType.DMA,
    ],
)


def _bn_relu_body(x_ref, w_ref, b_ref, o_ref):
    x = x_ref[...]
    ri = lax.broadcasted_iota(jnp.int32, (NPAD, 1), 0)
    msk = ri < N
    mean = jnp.sum(x, axis=0, keepdims=True) * (1.0 / N)
    d = x - mean
    dm = jnp.where(msk, d, 0.0)
    var = jnp.sum(dm * dm, axis=0, keepdims=True) * (1.0 / N)
    y = d * lax.rsqrt(var + EPS) * w_ref[...] + b_ref[...]
    y = jnp.maximum(y, 0.0)
    o_ref[pl.ds(0, NPAD), :] = jnp.where(msk, y, 0.0)
    o_ref[pl.ds(NPAD, XROWS - NPAD), :] = jnp.zeros((XROWS - NPAD, C),
                                                    jnp.float32)


def _bn_relu(xpad, w, b):
    return pl.pallas_call(
        _bn_relu_body,
        out_shape=jax.ShapeDtypeStruct((XROWS, C), jnp.float32),
    )(xpad, w.reshape(1, C), b.reshape(1, C))


def _mm_body(g_ref, w_ref, o_ref):
    k = pl.program_id(1)
    part = jnp.dot(g_ref[0], w_ref[0], preferred_element_type=jnp.float32)

    @pl.when(k == 0)
    def _():
        o_ref[...] = part

    @pl.when(k > 0)
    def _():
        o_ref[...] += part


def _mm_res_body(g_ref, w_ref, f_ref, o_ref):
    k = pl.program_id(1)
    part = jnp.dot(g_ref[0], w_ref[0], preferred_element_type=jnp.float32)

    @pl.when(k == 0)
    def _():
        o_ref[...] = f_ref[...] + part

    @pl.when(k > 0)
    def _():
        o_ref[...] += part


def _conv_mm(g, w):
    return pl.pallas_call(
        _mm_body,
        grid=(NRB, K27),
        in_specs=[
            pl.BlockSpec((1, RB, C), lambda j, k: (j, k, 0)),
            pl.BlockSpec((1, C, C), lambda j, k: (k, 0, 0)),
        ],
        out_specs=pl.BlockSpec((RB, C), lambda j, k: (j, 0)),
        out_shape=jax.ShapeDtypeStruct((NPAD, C), jnp.float32),
    )(g, w)


def _conv_mm_res(g, w, fpad):
    return pl.pallas_call(
        _mm_res_body,
        grid=(NRB, K27),
        in_specs=[
            pl.BlockSpec((1, RB, C), lambda j, k: (j, k, 0)),
            pl.BlockSpec((1, C, C), lambda j, k: (k, 0, 0)),
            pl.BlockSpec((RB, C), lambda j, k: (j, 0)),
        ],
        out_specs=pl.BlockSpec((RB, C), lambda j, k: (j, 0)),
        out_shape=jax.ShapeDtypeStruct((NPAD, C), jnp.float32),
    )(g, w, fpad)


def kernel(feat_list, pos_list, W1, W2, bn1_w, bn1_b, bn2_w, bn2_b):
    pos_i = pos_list.astype(jnp.int32)
    j = jnp.arange(NPAD - N, dtype=jnp.int32)
    pad_pos = jnp.stack([jnp.full_like(j, G), j // G, j % G], axis=1)
    posT = jnp.concatenate([pos_i, pad_pos], axis=0).T
    posx, posy, posz = posT[0], posT[1], posT[2]

    fidx = _nbr_kernel(posx, posy, posz)

    featpad = jnp.pad(feat_list, ((0, NPAD - N), (0, 0)))
    x1 = _bn_relu(featpad, bn1_w, bn1_b)
    g1 = _gather_kernel(x1, fidx)
    h = _conv_mm(g1, W1)
    x2 = _bn_relu(h, bn2_w, bn2_b)
    g2 = _gather_kernel(x2, fidx)
    outpad = _conv_mm_res(g2, W2, featpad)
    return outpad[:N]


# speedup vs baseline: 38.3855x; 2.0905x over previous
import functools

import jax
import jax.numpy as jnp
from jax import lax
from jax.experimental import pallas as pl
from jax.experimental.pallas import tpu as pltpu
from jax.experimental.pallas import tpu_sc as plsc

N = 10000
C = 128
G = 64
EPS = 1e-4

NC = 2
NS = 16
NW = NC * NS
CH = 320
NPAD = NW * CH
SENT = NPAD
XROWS = 10368
TBL = 294912
QCHUNKS = 68
RB = 320
NRB = NPAD // RB
K27 = 27

_mesh = plsc.VectorSubcoreMesh(
    core_axis_name="c", subcore_axis_name="s", num_cores=NC, num_subcores=NS)


def _nbr_body(posx, posy, posz, fidx_out,
              table, px_v, py_v, pz_v, key_v, val_v, qbuf, vbuf, tv, fo,
              neg_v, sem):
    c = lax.axis_index("c")
    s = lax.axis_index("s")
    wid = c * NS + s
    base = wid * CH

    def memset_loop(i, _):
        neg_v[pl.ds(i * 16, 16)] = jnp.full((16,), -1, jnp.int32)
        return 0
    lax.fori_loop(0, neg_v.shape[0] // 16, memset_loop, 0)
    seg = TBL // NS
    nfill = seg // neg_v.shape[0]
    for r in range(nfill):
        pltpu.sync_copy(neg_v, table.at[pl.ds(s * seg + r * neg_v.shape[0],
                                              neg_v.shape[0])])
    plsc.subcore_barrier()

    for half in range(2):
        cbase = (half * NS + s) * CH
        pltpu.sync_copy(posx.at[pl.ds(cbase, CH)], px_v)
        pltpu.sync_copy(posy.at[pl.ds(cbase, CH)], py_v)
        pltpu.sync_copy(posz.at[pl.ds(cbase, CH)], pz_v)

        def key_loop(i, _):
            x = px_v[pl.ds(i * 16, 16)]
            y = py_v[pl.ds(i * 16, 16)]
            z = pz_v[pl.ds(i * 16, 16)]
            k16 = (x * G + y) * G + z
            lane = lax.broadcasted_iota(jnp.int32, (16,), 0)
            r = i // 4
            col = (i % 4) * 16
            key_v[r, pl.ds(col, 16)] = k16
            val_v[r, pl.ds(col, 16)] = cbase + i * 16 + lane
            return 0
        lax.fori_loop(0, CH // 16, key_loop, 0)
        for r in range(CH // 64):
            pltpu.sync_copy(val_v.at[r], table.at[key_v.at[r]])
    plsc.subcore_barrier()

    for t in range(4):
        qbuf[QCHUNKS - 1, pl.ds(64 + t * 16, 16)] = jnp.zeros((16,),
                                                              jnp.int32)
    pltpu.sync_copy(posx.at[pl.ds(base, CH)], px_v)
    pltpu.sync_copy(posy.at[pl.ds(base, CH)], py_v)
    pltpu.sync_copy(posz.at[pl.ds(base, CH)], pz_v)

    def q_loop(koff, _):
        di = koff // 9 - 1
        dj = (koff // 3) % 3 - 1
        dk = koff % 3 - 1

        def q_inner(i, _):
            x = px_v[pl.ds(i * 16, 16)] + di
            y = py_v[pl.ds(i * 16, 16)] + dj
            z = pz_v[pl.ds(i * 16, 16)] + dk
            lane = lax.broadcasted_iota(jnp.int32, (16,), 0)
            nidx = base + i * 16 + lane
            inb = ((x >= 0) & (x < G) & (y >= 0) & (y < G)
                   & (z >= 0) & (z < G) & (nidx < N))
            qk = jnp.clip((x * G + y) * G + z, 0, TBL - 1)
            p = koff * CH + i * 16
            qbuf[p // 128, pl.ds(p % 128, 16)] = qk
            vbuf[pl.ds(p, 16)] = jnp.where(inb, 1, 0).astype(jnp.int32)
            return 0
        lax.fori_loop(0, CH // 16, q_inner, 0)
        return 0
    lax.fori_loop(0, K27, q_loop, 0)

    def g_loop(j, _):
        pltpu.sync_copy(table.at[qbuf.at[j]], tv.at[j])
        return 0
    lax.fori_loop(0, QCHUNKS, g_loop, 0)

    def f_loop(i, _):
        p = i * 16
        t = tv[p // 128, pl.ds(p % 128, 16)]
        v = vbuf[pl.ds(p, 16)]
        found = (t >= 0) & (v > 0)
        fo[pl.ds(p, 16)] = jnp.where(found, t, SENT).astype(jnp.int32)
        return 0
    lax.fori_loop(0, (K27 * CH) // 16, f_loop, 0)

    pltpu.sync_copy(fo, fidx_out.at[pl.ds(wid * K27 * CH, K27 * CH)])


_nbr_kernel = pl.kernel(
    _nbr_body,
    out_type=jax.ShapeDtypeStruct((NW * K27 * CH,), jnp.int32),
    mesh=_mesh,
    scratch_types=[
        pltpu.VMEM_SHARED((TBL,), jnp.int32),
        pltpu.VMEM((CH,), jnp.int32),
        pltpu.VMEM((CH,), jnp.int32),
        pltpu.VMEM((CH,), jnp.int32),
        pltpu.VMEM((CH // 64, 64), jnp.int32),
        pltpu.VMEM((CH // 64, 64), jnp.int32),
        pltpu.VMEM((QCHUNKS, 128), jnp.int32),
        pltpu.VMEM((QCHUNKS * 128,), jnp.int32),
        pltpu.VMEM((QCHUNKS, 128), jnp.int32),
        pltpu.VMEM((K27 * CH,), jnp.int32),
        pltpu.VMEM((2048,), jnp.int32),
        pltpu.SemaphoreType.DMA,
    ],
)


GCHUNK = 120
NGCH = (K27 * CH) // GCHUNK


def _gather_body(xpad, fidx, g_out, xsh, idx_v, rows_a, rows_b, sem_g, sem_w):
    c = lax.axis_index("c")
    s = lax.axis_index("s")
    wid = c * NS + s
    gbase = wid * K27 * CH

    cps = XROWS // NS
    off = 0
    for ln in (120, 120, 120, 120, 120, 48):
        r0 = s * cps + off
        pltpu.sync_copy(xpad.at[pl.ds(r0, ln)], rows_a.at[pl.ds(0, ln)])
        pltpu.sync_copy(rows_a.at[pl.ds(0, ln)], xsh.at[pl.ds(r0, ln)])
        off += ln
    plsc.subcore_barrier()

    pltpu.sync_copy(fidx.at[pl.ds(gbase, K27 * CH)], idx_v)
    bufs = (rows_a, rows_b)
    gd = [None] * NGCH
    wd = [None] * NGCH
    for ci in range(NGCH):
        if ci >= 2:
            wd[ci - 2].wait()
        gd[ci] = pltpu.async_copy(
            xsh.at[idx_v.at[pl.ds(ci * GCHUNK, GCHUNK)]],
            bufs[ci % 2], sem_g)
        if ci >= 1:
            gd[ci - 1].wait()
            wd[ci - 1] = pltpu.async_copy(
                bufs[(ci - 1) % 2],
                g_out.at[wid, pl.ds((ci - 1) * GCHUNK, GCHUNK)], sem_w)
    gd[NGCH - 1].wait()
    wd[NGCH - 1] = pltpu.async_copy(
        bufs[(NGCH - 1) % 2],
        g_out.at[wid, pl.ds((NGCH - 1) * GCHUNK, GCHUNK)], sem_w)
    wd[NGCH - 2].wait()
    wd[NGCH - 1].wait()


_gather_kernel = pl.kernel(
    _gather_body,
    out_type=jax.ShapeDtypeStruct((NW, K27 * CH, C), jnp.float32),
    mesh=_mesh,
    scratch_types=[
        pltpu.VMEM_SHARED((XROWS, C), jnp.float32),
        pltpu.VMEM((K27 * CH,), jnp.int32),
        pltpu.VMEM((GCHUNK, C), jnp.float32),
        pltpu.VMEM((GCHUNK, C), jnp.float32),
        pltpu.SemaphoreType.DMA,
        pltpu.SemaphoreType.DMA,
    ],
)


def _bn_relu_body(x_ref, w_ref, b_ref, o_ref):
    x = x_ref[...]
    ri = lax.broadcasted_iota(jnp.int32, (NPAD, 1), 0)
    msk = ri < N
    mean = jnp.sum(x, axis=0, keepdims=True) * (1.0 / N)
    d = x - mean
    dm = jnp.where(msk, d, 0.0)
    var = jnp.sum(dm * dm, axis=0, keepdims=True) * (1.0 / N)
    y = d * lax.rsqrt(var + EPS) * w_ref[...] + b_ref[...]
    y = jnp.maximum(y, 0.0)
    o_ref[pl.ds(0, NPAD), :] = jnp.where(msk, y, 0.0)
    o_ref[pl.ds(NPAD, XROWS - NPAD), :] = jnp.zeros((XROWS - NPAD, C),
                                                    jnp.float32)


def _bn_relu(xpad, w, b):
    return pl.pallas_call(
        _bn_relu_body,
        out_shape=jax.ShapeDtypeStruct((XROWS, C), jnp.float32),
    )(xpad, w.reshape(1, C), b.reshape(1, C))


BW = 4
MR = BW * RB


def _mm_body(g_ref, w_ref, o_ref):
    k = pl.program_id(1)
    gm = g_ref[...].reshape(MR, C)
    part = jnp.dot(gm, w_ref[0], preferred_element_type=jnp.float32)

    @pl.when(k == 0)
    def _():
        o_ref[...] = part

    @pl.when(k > 0)
    def _():
        o_ref[...] += part


def _mm_res_body(g_ref, w_ref, f_ref, o_ref):
    k = pl.program_id(1)
    gm = g_ref[...].reshape(MR, C)
    part = jnp.dot(gm, w_ref[0], preferred_element_type=jnp.float32)

    @pl.when(k == 0)
    def _():
        o_ref[...] = f_ref[...] + part

    @pl.when(k > 0)
    def _():
        o_ref[...] += part


def _conv_mm(g, w):
    return pl.pallas_call(
        _mm_body,
        grid=(NPAD // MR, K27),
        in_specs=[
            pl.BlockSpec((BW, RB, C), lambda j, k: (j, k, 0)),
            pl.BlockSpec((1, C, C), lambda j, k: (k, 0, 0)),
        ],
        out_specs=pl.BlockSpec((MR, C), lambda j, k: (j, 0)),
        out_shape=jax.ShapeDtypeStruct((NPAD, C), jnp.float32),
    )(g, w)


def _conv_mm_res(g, w, fpad):
    return pl.pallas_call(
        _mm_res_body,
        grid=(NPAD // MR, K27),
        in_specs=[
            pl.BlockSpec((BW, RB, C), lambda j, k: (j, k, 0)),
            pl.BlockSpec((1, C, C), lambda j, k: (k, 0, 0)),
            pl.BlockSpec((MR, C), lambda j, k: (j, 0)),
        ],
        out_specs=pl.BlockSpec((MR, C), lambda j, k: (j, 0)),
        out_shape=jax.ShapeDtypeStruct((NPAD, C), jnp.float32),
    )(g, w, fpad)


def kernel(feat_list, pos_list, W1, W2, bn1_w, bn1_b, bn2_w, bn2_b):
    pos_i = pos_list.astype(jnp.int32)
    j = jnp.arange(NPAD - N, dtype=jnp.int32)
    pad_pos = jnp.stack([jnp.full_like(j, G), j // G, j % G], axis=1)
    posT = jnp.concatenate([pos_i, pad_pos], axis=0).T
    posx, posy, posz = posT[0], posT[1], posT[2]

    fidx = _nbr_kernel(posx, posy, posz)

    featpad = jnp.pad(feat_list, ((0, NPAD - N), (0, 0)))
    x1 = _bn_relu(featpad, bn1_w, bn1_b)
    g1 = _gather_kernel(x1, fidx)
    h = _conv_mm(g1, W1)
    x2 = _bn_relu(h, bn2_w, bn2_b)
    g2 = _gather_kernel(x2, fidx)
    outpad = _conv_mm_res(g2, W2, featpad)
    return outpad[:N]


# speedup vs baseline: 47.3604x; 1.2338x over previous
import functools

import jax
import jax.numpy as jnp
from jax import lax
from jax.experimental import pallas as pl
from jax.experimental.pallas import tpu as pltpu
from jax.experimental.pallas import tpu_sc as plsc

N = 10000
C = 128
G = 64
EPS = 1e-4

NC = 2
NS = 16
NW = NC * NS
CH = 320
NPAD = NW * CH
SENT = NPAD
XROWS = 10368
TBL = 294912
QCHUNKS = 68
RB = 320
NRB = NPAD // RB
K27 = 27

_mesh = plsc.VectorSubcoreMesh(
    core_axis_name="c", subcore_axis_name="s", num_cores=NC, num_subcores=NS)


def _nbr_body(posx, posy, posz, fidx_out,
              table, px_v, py_v, pz_v, key_v, val_v, qbuf, vbuf, tv, fo,
              neg_v, sem):
    c = lax.axis_index("c")
    s = lax.axis_index("s")
    wid = c * NS + s
    base = wid * CH

    def memset_loop(i, _):
        neg_v[pl.ds(i * 16, 16)] = jnp.full((16,), -1, jnp.int32)
        return 0
    lax.fori_loop(0, neg_v.shape[0] // 16, memset_loop, 0)
    seg = TBL // NS
    nfill = seg // neg_v.shape[0]
    for r in range(nfill):
        pltpu.sync_copy(neg_v, table.at[pl.ds(s * seg + r * neg_v.shape[0],
                                              neg_v.shape[0])])
    plsc.subcore_barrier()

    for half in range(2):
        cbase = (half * NS + s) * CH
        pltpu.sync_copy(posx.at[pl.ds(cbase, CH)], px_v)
        pltpu.sync_copy(posy.at[pl.ds(cbase, CH)], py_v)
        pltpu.sync_copy(posz.at[pl.ds(cbase, CH)], pz_v)

        def key_loop(i, _):
            x = px_v[pl.ds(i * 16, 16)]
            y = py_v[pl.ds(i * 16, 16)]
            z = pz_v[pl.ds(i * 16, 16)]
            k16 = (x * G + y) * G + z
            lane = lax.broadcasted_iota(jnp.int32, (16,), 0)
            r = i // 4
            col = (i % 4) * 16
            key_v[r, pl.ds(col, 16)] = k16
            val_v[r, pl.ds(col, 16)] = cbase + i * 16 + lane
            return 0
        lax.fori_loop(0, CH // 16, key_loop, 0)
        for r in range(CH // 64):
            pltpu.sync_copy(val_v.at[r], table.at[key_v.at[r]])
    plsc.subcore_barrier()

    for t in range(4):
        qbuf[QCHUNKS - 1, pl.ds(64 + t * 16, 16)] = jnp.zeros((16,),
                                                              jnp.int32)
    pltpu.sync_copy(posx.at[pl.ds(base, CH)], px_v)
    pltpu.sync_copy(posy.at[pl.ds(base, CH)], py_v)
    pltpu.sync_copy(posz.at[pl.ds(base, CH)], pz_v)

    def q_loop(koff, _):
        di = koff // 9 - 1
        dj = (koff // 3) % 3 - 1
        dk = koff % 3 - 1

        def q_inner(i, _):
            x = px_v[pl.ds(i * 16, 16)] + di
            y = py_v[pl.ds(i * 16, 16)] + dj
            z = pz_v[pl.ds(i * 16, 16)] + dk
            lane = lax.broadcasted_iota(jnp.int32, (16,), 0)
            nidx = base + i * 16 + lane
            inb = ((x >= 0) & (x < G) & (y >= 0) & (y < G)
                   & (z >= 0) & (z < G) & (nidx < N))
            qk = jnp.clip((x * G + y) * G + z, 0, TBL - 1)
            p = koff * CH + i * 16
            qbuf[p // 128, pl.ds(p % 128, 16)] = qk
            vbuf[pl.ds(p, 16)] = jnp.where(inb, 1, 0).astype(jnp.int32)
            return 0
        lax.fori_loop(0, CH // 16, q_inner, 0)
        return 0
    lax.fori_loop(0, K27, q_loop, 0)

    def g_loop(j, _):
        pltpu.sync_copy(table.at[qbuf.at[j]], tv.at[j])
        return 0
    lax.fori_loop(0, QCHUNKS, g_loop, 0)

    def f_loop(i, _):
        p = i * 16
        t = tv[p // 128, pl.ds(p % 128, 16)]
        v = vbuf[pl.ds(p, 16)]
        found = (t >= 0) & (v > 0)
        fo[pl.ds(p, 16)] = jnp.where(found, t, SENT).astype(jnp.int32)
        return 0
    lax.fori_loop(0, (K27 * CH) // 16, f_loop, 0)

    pltpu.sync_copy(fo, fidx_out.at[pl.ds(wid * K27 * CH, K27 * CH)])


_nbr_kernel = pl.kernel(
    _nbr_body,
    out_type=jax.ShapeDtypeStruct((NW * K27 * CH,), jnp.int32),
    mesh=_mesh,
    scratch_types=[
        pltpu.VMEM_SHARED((TBL,), jnp.int32),
        pltpu.VMEM((CH,), jnp.int32),
        pltpu.VMEM((CH,), jnp.int32),
        pltpu.VMEM((CH,), jnp.int32),
        pltpu.VMEM((CH // 64, 64), jnp.int32),
        pltpu.VMEM((CH // 64, 64), jnp.int32),
        pltpu.VMEM((QCHUNKS, 128), jnp.int32),
        pltpu.VMEM((QCHUNKS * 128,), jnp.int32),
        pltpu.VMEM((QCHUNKS, 128), jnp.int32),
        pltpu.VMEM((K27 * CH,), jnp.int32),
        pltpu.VMEM((2048,), jnp.int32),
        pltpu.SemaphoreType.DMA,
    ],
)


GCHUNK = 120
NGCH = (K27 * CH) // GCHUNK


def _gather_body(xpad, fidx, g_out, xsh, idx_v, rows_a, rows_b, sem_g, sem_w):
    c = lax.axis_index("c")
    s = lax.axis_index("s")
    wid = c * NS + s
    gbase = wid * K27 * CH

    cps = XROWS // NS
    off = 0
    for ln in (120, 120, 120, 120, 120, 48):
        r0 = s * cps + off
        pltpu.sync_copy(xpad.at[pl.ds(r0, ln)], rows_a.at[pl.ds(0, ln)])
        pltpu.sync_copy(rows_a.at[pl.ds(0, ln)], xsh.at[pl.ds(r0, ln)])
        off += ln
    plsc.subcore_barrier()

    pltpu.sync_copy(fidx.at[pl.ds(gbase, K27 * CH)], idx_v)
    bufs = (rows_a, rows_b)
    gd = [None] * NGCH
    wd = [None] * NGCH
    for ci in range(NGCH):
        if ci >= 2:
            wd[ci - 2].wait()
        gd[ci] = pltpu.async_copy(
            xsh.at[idx_v.at[pl.ds(ci * GCHUNK, GCHUNK)]],
            bufs[ci % 2], sem_g)
        if ci >= 1:
            gd[ci - 1].wait()
            wd[ci - 1] = pltpu.async_copy(
                bufs[(ci - 1) % 2],
                g_out.at[wid, pl.ds((ci - 1) * GCHUNK, GCHUNK)], sem_w)
    gd[NGCH - 1].wait()
    wd[NGCH - 1] = pltpu.async_copy(
        bufs[(NGCH - 1) % 2],
        g_out.at[wid, pl.ds((NGCH - 1) * GCHUNK, GCHUNK)], sem_w)
    wd[NGCH - 2].wait()
    wd[NGCH - 1].wait()


_gather_kernel = pl.kernel(
    _gather_body,
    out_type=jax.ShapeDtypeStruct((NW, K27 * CH, C), jnp.float32),
    mesh=_mesh,
    scratch_types=[
        pltpu.VMEM_SHARED((XROWS, C), jnp.float32),
        pltpu.VMEM((K27 * CH,), jnp.int32),
        pltpu.VMEM((GCHUNK, C), jnp.float32),
        pltpu.VMEM((GCHUNK, C), jnp.float32),
        pltpu.SemaphoreType.DMA,
        pltpu.SemaphoreType.DMA,
    ],
)


def _bn_relu_body(x_ref, w_ref, b_ref, o_ref):
    x = x_ref[...]
    ri = lax.broadcasted_iota(jnp.int32, (NPAD, 1), 0)
    msk = ri < N
    mean = jnp.sum(x, axis=0, keepdims=True) * (1.0 / N)
    d = x - mean
    dm = jnp.where(msk, d, 0.0)
    var = jnp.sum(dm * dm, axis=0, keepdims=True) * (1.0 / N)
    y = d * lax.rsqrt(var + EPS) * w_ref[...] + b_ref[...]
    y = jnp.maximum(y, 0.0)
    o_ref[pl.ds(0, NPAD), :] = jnp.where(msk, y, 0.0)
    o_ref[pl.ds(NPAD, XROWS - NPAD), :] = jnp.zeros((XROWS - NPAD, C),
                                                    jnp.float32)


def _bn_relu(xpad, w, b):
    return pl.pallas_call(
        _bn_relu_body,
        out_shape=jax.ShapeDtypeStruct((XROWS, C), jnp.float32),
    )(xpad, w.reshape(1, C), b.reshape(1, C))


BW = 8
MR = BW * RB


def _mm_body(g_ref, w_ref, o_ref):
    k = pl.program_id(1)
    gm = g_ref[...].reshape(MR, C)
    part = jnp.dot(gm, w_ref[0], preferred_element_type=jnp.float32)

    @pl.when(k == 0)
    def _():
        o_ref[...] = part

    @pl.when(k > 0)
    def _():
        o_ref[...] += part


def _mm_res_body(g_ref, w_ref, f_ref, o_ref):
    k = pl.program_id(1)
    gm = g_ref[...].reshape(MR, C)
    part = jnp.dot(gm, w_ref[0], preferred_element_type=jnp.float32)

    @pl.when(k == 0)
    def _():
        o_ref[...] = f_ref[...] + part

    @pl.when(k > 0)
    def _():
        o_ref[...] += part


def _conv_mm(g, w):
    return pl.pallas_call(
        _mm_body,
        grid=(NPAD // MR, K27),
        in_specs=[
            pl.BlockSpec((BW, RB, C), lambda j, k: (j, k, 0)),
            pl.BlockSpec((1, C, C), lambda j, k: (k, 0, 0)),
        ],
        out_specs=pl.BlockSpec((MR, C), lambda j, k: (j, 0)),
        out_shape=jax.ShapeDtypeStruct((NPAD, C), jnp.float32),
    )(g, w)


def _conv_mm_res(g, w, fpad):
    return pl.pallas_call(
        _mm_res_body,
        grid=(NPAD // MR, K27),
        in_specs=[
            pl.BlockSpec((BW, RB, C), lambda j, k: (j, k, 0)),
            pl.BlockSpec((1, C, C), lambda j, k: (k, 0, 0)),
            pl.BlockSpec((MR, C), lambda j, k: (j, 0)),
        ],
        out_specs=pl.BlockSpec((MR, C), lambda j, k: (j, 0)),
        out_shape=jax.ShapeDtypeStruct((NPAD, C), jnp.float32),
    )(g, w, fpad)


def kernel(feat_list, pos_list, W1, W2, bn1_w, bn1_b, bn2_w, bn2_b):
    pos_i = pos_list.astype(jnp.int32)
    j = jnp.arange(NPAD - N, dtype=jnp.int32)
    pad_pos = jnp.stack([jnp.full_like(j, G), j // G, j % G], axis=1)
    posT = jnp.concatenate([pos_i, pad_pos], axis=0).T
    posx, posy, posz = posT[0], posT[1], posT[2]

    fidx = _nbr_kernel(posx, posy, posz)

    featpad = jnp.pad(feat_list, ((0, NPAD - N), (0, 0)))
    x1 = _bn_relu(featpad, bn1_w, bn1_b)
    g1 = _gather_kernel(x1, fidx)
    h = _conv_mm(g1, W1)
    x2 = _bn_relu(h, bn2_w, bn2_b)
    g2 = _gather_kernel(x2, fidx)
    outpad = _conv_mm_res(g2, W2, featpad)
    return outpad[:N]


# speedup vs baseline: 53.9547x; 1.1392x over previous
import functools

import jax
import jax.numpy as jnp
from jax import lax
from jax.experimental import pallas as pl
from jax.experimental.pallas import tpu as pltpu
from jax.experimental.pallas import tpu_sc as plsc

N = 10000
C = 128
G = 64
EPS = 1e-4

NC = 2
NS = 16
NW = NC * NS
CH = 320
NPAD = NW * CH
SENT = NPAD
XROWS = 10368
TBL = 294912
QCHUNKS = 68
RB = 320
NRB = NPAD // RB
K27 = 27

_mesh = plsc.VectorSubcoreMesh(
    core_axis_name="c", subcore_axis_name="s", num_cores=NC, num_subcores=NS)


def _nbr_body(posx, posy, posz, fidx_out,
              table, px_v, py_v, pz_v, key_v, val_v, qbuf, vbuf, tv, fo,
              neg_v, sem):
    c = lax.axis_index("c")
    s = lax.axis_index("s")
    wid = c * NS + s
    base = wid * CH

    def memset_loop(i, _):
        neg_v[pl.ds(i * 16, 16)] = jnp.full((16,), -1, jnp.int32)
        return 0
    lax.fori_loop(0, neg_v.shape[0] // 16, memset_loop, 0)
    seg = TBL // NS
    nfill = seg // neg_v.shape[0]
    for r in range(nfill):
        pltpu.sync_copy(neg_v, table.at[pl.ds(s * seg + r * neg_v.shape[0],
                                              neg_v.shape[0])])
    plsc.subcore_barrier()

    for half in range(2):
        cbase = (half * NS + s) * CH
        pltpu.sync_copy(posx.at[pl.ds(cbase, CH)], px_v)
        pltpu.sync_copy(posy.at[pl.ds(cbase, CH)], py_v)
        pltpu.sync_copy(posz.at[pl.ds(cbase, CH)], pz_v)

        def key_loop(i, _):
            x = px_v[pl.ds(i * 16, 16)]
            y = py_v[pl.ds(i * 16, 16)]
            z = pz_v[pl.ds(i * 16, 16)]
            k16 = (x * G + y) * G + z
            lane = lax.broadcasted_iota(jnp.int32, (16,), 0)
            r = i // 4
            col = (i % 4) * 16
            key_v[r, pl.ds(col, 16)] = k16
            val_v[r, pl.ds(col, 16)] = cbase + i * 16 + lane
            return 0
        lax.fori_loop(0, CH // 16, key_loop, 0)
        for r in range(CH // 64):
            pltpu.sync_copy(val_v.at[r], table.at[key_v.at[r]])
    plsc.subcore_barrier()

    for t in range(4):
        qbuf[QCHUNKS - 1, pl.ds(64 + t * 16, 16)] = jnp.zeros((16,),
                                                              jnp.int32)
    pltpu.sync_copy(posx.at[pl.ds(base, CH)], px_v)
    pltpu.sync_copy(posy.at[pl.ds(base, CH)], py_v)
    pltpu.sync_copy(posz.at[pl.ds(base, CH)], pz_v)

    def q_loop(koff, _):
        di = koff // 9 - 1
        dj = (koff // 3) % 3 - 1
        dk = koff % 3 - 1

        def q_inner(i, _):
            x = px_v[pl.ds(i * 16, 16)] + di
            y = py_v[pl.ds(i * 16, 16)] + dj
            z = pz_v[pl.ds(i * 16, 16)] + dk
            lane = lax.broadcasted_iota(jnp.int32, (16,), 0)
            nidx = base + i * 16 + lane
            inb = ((x >= 0) & (x < G) & (y >= 0) & (y < G)
                   & (z >= 0) & (z < G) & (nidx < N))
            qk = jnp.clip((x * G + y) * G + z, 0, TBL - 1)
            p = koff * CH + i * 16
            qbuf[p // 128, pl.ds(p % 128, 16)] = qk
            vbuf[pl.ds(p, 16)] = jnp.where(inb, 1, 0).astype(jnp.int32)
            return 0
        lax.fori_loop(0, CH // 16, q_inner, 0)
        return 0
    lax.fori_loop(0, K27, q_loop, 0)

    def g_loop(j, _):
        pltpu.sync_copy(table.at[qbuf.at[j]], tv.at[j])
        return 0
    lax.fori_loop(0, QCHUNKS, g_loop, 0)

    def f_loop(i, _):
        p = i * 16
        t = tv[p // 128, pl.ds(p % 128, 16)]
        v = vbuf[pl.ds(p, 16)]
        found = (t >= 0) & (v > 0)
        fo[pl.ds(p, 16)] = jnp.where(found, t, SENT).astype(jnp.int32)
        return 0
    lax.fori_loop(0, (K27 * CH) // 16, f_loop, 0)

    pltpu.sync_copy(fo, fidx_out.at[pl.ds(wid * K27 * CH, K27 * CH)])


_nbr_kernel = pl.kernel(
    _nbr_body,
    out_type=jax.ShapeDtypeStruct((NW * K27 * CH,), jnp.int32),
    mesh=_mesh,
    scratch_types=[
        pltpu.VMEM_SHARED((TBL,), jnp.int32),
        pltpu.VMEM((CH,), jnp.int32),
        pltpu.VMEM((CH,), jnp.int32),
        pltpu.VMEM((CH,), jnp.int32),
        pltpu.VMEM((CH // 64, 64), jnp.int32),
        pltpu.VMEM((CH // 64, 64), jnp.int32),
        pltpu.VMEM((QCHUNKS, 128), jnp.int32),
        pltpu.VMEM((QCHUNKS * 128,), jnp.int32),
        pltpu.VMEM((QCHUNKS, 128), jnp.int32),
        pltpu.VMEM((K27 * CH,), jnp.int32),
        pltpu.VMEM((2048,), jnp.int32),
        pltpu.SemaphoreType.DMA,
    ],
)


GCHUNK = 120
NGCH = (K27 * CH) // GCHUNK


def _gather_body(xpad, fidx, g_out, xsh, idx_v, rows_a, rows_b, sem_g, sem_w):
    c = lax.axis_index("c")
    s = lax.axis_index("s")
    wid = c * NS + s
    gbase = wid * K27 * CH

    cps = XROWS // NS
    off = 0
    for ln in (120, 120, 120, 120, 120, 48):
        r0 = s * cps + off
        pltpu.sync_copy(xpad.at[pl.ds(r0, ln)], rows_a.at[pl.ds(0, ln)])
        pltpu.sync_copy(rows_a.at[pl.ds(0, ln)], xsh.at[pl.ds(r0, ln)])
        off += ln
    plsc.subcore_barrier()

    pltpu.sync_copy(fidx.at[pl.ds(gbase, K27 * CH)], idx_v)
    bufs = (rows_a, rows_b)
    gd = [None] * NGCH
    wd = [None] * NGCH
    for ci in range(NGCH):
        if ci >= 2:
            wd[ci - 2].wait()
        gd[ci] = pltpu.async_copy(
            xsh.at[idx_v.at[pl.ds(ci * GCHUNK, GCHUNK)]],
            bufs[ci % 2], sem_g)
        if ci >= 1:
            gd[ci - 1].wait()
            wd[ci - 1] = pltpu.async_copy(
                bufs[(ci - 1) % 2],
                g_out.at[wid, pl.ds((ci - 1) * GCHUNK, GCHUNK)], sem_w)
    gd[NGCH - 1].wait()
    wd[NGCH - 1] = pltpu.async_copy(
        bufs[(NGCH - 1) % 2],
        g_out.at[wid, pl.ds((NGCH - 1) * GCHUNK, GCHUNK)], sem_w)
    wd[NGCH - 2].wait()
    wd[NGCH - 1].wait()


_gather_kernel = pl.kernel(
    _gather_body,
    out_type=jax.ShapeDtypeStruct((NW, K27 * CH, C), jnp.float32),
    mesh=_mesh,
    scratch_types=[
        pltpu.VMEM_SHARED((XROWS, C), jnp.float32),
        pltpu.VMEM((K27 * CH,), jnp.int32),
        pltpu.VMEM((GCHUNK, C), jnp.float32),
        pltpu.VMEM((GCHUNK, C), jnp.float32),
        pltpu.SemaphoreType.DMA,
        pltpu.SemaphoreType.DMA,
    ],
)


def _bn_relu_body(x_ref, w_ref, b_ref, o_ref):
    x = x_ref[...]
    ri = lax.broadcasted_iota(jnp.int32, (NPAD, 1), 0)
    msk = ri < N
    mean = jnp.sum(x, axis=0, keepdims=True) * (1.0 / N)
    d = x - mean
    dm = jnp.where(msk, d, 0.0)
    var = jnp.sum(dm * dm, axis=0, keepdims=True) * (1.0 / N)
    y = d * lax.rsqrt(var + EPS) * w_ref[...] + b_ref[...]
    y = jnp.maximum(y, 0.0)
    o_ref[pl.ds(0, NPAD), :] = jnp.where(msk, y, 0.0)
    o_ref[pl.ds(NPAD, XROWS - NPAD), :] = jnp.zeros((XROWS - NPAD, C),
                                                    jnp.float32)


def _bn_relu(xpad, w, b):
    return pl.pallas_call(
        _bn_relu_body,
        out_shape=jax.ShapeDtypeStruct((XROWS, C), jnp.float32),
    )(xpad, w.reshape(1, C), b.reshape(1, C))


BW = 16
MR = BW * RB


def _mm_body(g_ref, w_ref, o_ref):
    k = pl.program_id(1)
    gm = g_ref[...].reshape(MR, C)
    part = jnp.dot(gm, w_ref[0], preferred_element_type=jnp.float32)

    @pl.when(k == 0)
    def _():
        o_ref[...] = part

    @pl.when(k > 0)
    def _():
        o_ref[...] += part


def _mm_res_body(g_ref, w_ref, f_ref, o_ref):
    k = pl.program_id(1)
    gm = g_ref[...].reshape(MR, C)
    part = jnp.dot(gm, w_ref[0], preferred_element_type=jnp.float32)

    @pl.when(k == 0)
    def _():
        o_ref[...] = f_ref[...] + part

    @pl.when(k > 0)
    def _():
        o_ref[...] += part


def _conv_mm(g, w):
    return pl.pallas_call(
        _mm_body,
        grid=(NPAD // MR, K27),
        in_specs=[
            pl.BlockSpec((BW, RB, C), lambda j, k: (j, k, 0)),
            pl.BlockSpec((1, C, C), lambda j, k: (k, 0, 0)),
        ],
        out_specs=pl.BlockSpec((MR, C), lambda j, k: (j, 0)),
        out_shape=jax.ShapeDtypeStruct((NPAD, C), jnp.float32),
    )(g, w)


def _conv_mm_res(g, w, fpad):
    return pl.pallas_call(
        _mm_res_body,
        grid=(NPAD // MR, K27),
        in_specs=[
            pl.BlockSpec((BW, RB, C), lambda j, k: (j, k, 0)),
            pl.BlockSpec((1, C, C), lambda j, k: (k, 0, 0)),
            pl.BlockSpec((MR, C), lambda j, k: (j, 0)),
        ],
        out_specs=pl.BlockSpec((MR, C), lambda j, k: (j, 0)),
        out_shape=jax.ShapeDtypeStruct((NPAD, C), jnp.float32),
    )(g, w, fpad)


def kernel(feat_list, pos_list, W1, W2, bn1_w, bn1_b, bn2_w, bn2_b):
    pos_i = pos_list.astype(jnp.int32)
    j = jnp.arange(NPAD - N, dtype=jnp.int32)
    pad_pos = jnp.stack([jnp.full_like(j, G), j // G, j % G], axis=1)
    posT = jnp.concatenate([pos_i, pad_pos], axis=0).T
    posx, posy, posz = posT[0], posT[1], posT[2]

    fidx = _nbr_kernel(posx, posy, posz)

    featpad = jnp.pad(feat_list, ((0, NPAD - N), (0, 0)))
    x1 = _bn_relu(featpad, bn1_w, bn1_b)
    g1 = _gather_kernel(x1, fidx)
    h = _conv_mm(g1, W1)
    x2 = _bn_relu(h, bn2_w, bn2_b)
    g2 = _gather_kernel(x2, fidx)
    outpad = _conv_mm_res(g2, W2, featpad)
    return outpad[:N]


# speedup vs baseline: 58.1373x; 1.0775x over previous
import functools

import jax
import jax.numpy as jnp
from jax import lax
from jax.experimental import pallas as pl
from jax.experimental.pallas import tpu as pltpu
from jax.experimental.pallas import tpu_sc as plsc

N = 10000
C = 128
G = 64
EPS = 1e-4

NC = 2
NS = 16
NW = NC * NS
CH = 320
NPAD = NW * CH
SENT = NPAD
XROWS = 10368
TBL = 294912
QCHUNKS = 68
RB = 320
NRB = NPAD // RB
K27 = 27

_mesh = plsc.VectorSubcoreMesh(
    core_axis_name="c", subcore_axis_name="s", num_cores=NC, num_subcores=NS)


def _nbr_body(posx, posy, posz, fidx_out,
              table, px_v, py_v, pz_v, key_v, val_v, qbuf, vbuf, tv, fo,
              neg_v, sem):
    c = lax.axis_index("c")
    s = lax.axis_index("s")
    wid = c * NS + s
    base = wid * CH

    def memset_loop(i, _):
        neg_v[pl.ds(i * 16, 16)] = jnp.full((16,), -1, jnp.int32)
        return 0
    lax.fori_loop(0, neg_v.shape[0] // 16, memset_loop, 0)
    seg = TBL // NS
    nfill = seg // neg_v.shape[0]
    for r in range(nfill):
        pltpu.sync_copy(neg_v, table.at[pl.ds(s * seg + r * neg_v.shape[0],
                                              neg_v.shape[0])])
    plsc.subcore_barrier()

    for half in range(2):
        cbase = (half * NS + s) * CH
        pltpu.sync_copy(posx.at[pl.ds(cbase, CH)], px_v)
        pltpu.sync_copy(posy.at[pl.ds(cbase, CH)], py_v)
        pltpu.sync_copy(posz.at[pl.ds(cbase, CH)], pz_v)

        def key_loop(i, _):
            x = px_v[pl.ds(i * 16, 16)]
            y = py_v[pl.ds(i * 16, 16)]
            z = pz_v[pl.ds(i * 16, 16)]
            k16 = (x * G + y) * G + z
            lane = lax.broadcasted_iota(jnp.int32, (16,), 0)
            r = i // 4
            col = (i % 4) * 16
            key_v[r, pl.ds(col, 16)] = k16
            val_v[r, pl.ds(col, 16)] = cbase + i * 16 + lane
            return 0
        lax.fori_loop(0, CH // 16, key_loop, 0)
        for r in range(CH // 64):
            pltpu.sync_copy(val_v.at[r], table.at[key_v.at[r]])
    plsc.subcore_barrier()

    for t in range(4):
        qbuf[QCHUNKS - 1, pl.ds(64 + t * 16, 16)] = jnp.zeros((16,),
                                                              jnp.int32)
    pltpu.sync_copy(posx.at[pl.ds(base, CH)], px_v)
    pltpu.sync_copy(posy.at[pl.ds(base, CH)], py_v)
    pltpu.sync_copy(posz.at[pl.ds(base, CH)], pz_v)

    def q_loop(koff, _):
        di = koff // 9 - 1
        dj = (koff // 3) % 3 - 1
        dk = koff % 3 - 1

        def q_inner(i, _):
            x = px_v[pl.ds(i * 16, 16)] + di
            y = py_v[pl.ds(i * 16, 16)] + dj
            z = pz_v[pl.ds(i * 16, 16)] + dk
            lane = lax.broadcasted_iota(jnp.int32, (16,), 0)
            nidx = base + i * 16 + lane
            inb = ((x >= 0) & (x < G) & (y >= 0) & (y < G)
                   & (z >= 0) & (z < G) & (nidx < N))
            qk = jnp.clip((x * G + y) * G + z, 0, TBL - 1)
            p = koff * CH + i * 16
            qbuf[p // 128, pl.ds(p % 128, 16)] = qk
            vbuf[pl.ds(p, 16)] = jnp.where(inb, 1, 0).astype(jnp.int32)
            return 0
        lax.fori_loop(0, CH // 16, q_inner, 0)
        return 0
    lax.fori_loop(0, K27, q_loop, 0)

    def g_loop(j, _):
        pltpu.sync_copy(table.at[qbuf.at[j]], tv.at[j])
        return 0
    lax.fori_loop(0, QCHUNKS, g_loop, 0)

    def f_loop(i, _):
        p = i * 16
        t = tv[p // 128, pl.ds(p % 128, 16)]
        v = vbuf[pl.ds(p, 16)]
        found = (t >= 0) & (v > 0)
        fo[pl.ds(p, 16)] = jnp.where(found, t, SENT).astype(jnp.int32)
        return 0
    lax.fori_loop(0, (K27 * CH) // 16, f_loop, 0)

    pltpu.sync_copy(fo, fidx_out.at[pl.ds(wid * K27 * CH, K27 * CH)])


_nbr_kernel = pl.kernel(
    _nbr_body,
    out_type=jax.ShapeDtypeStruct((NW * K27 * CH,), jnp.int32),
    mesh=_mesh,
    scratch_types=[
        pltpu.VMEM_SHARED((TBL,), jnp.int32),
        pltpu.VMEM((CH,), jnp.int32),
        pltpu.VMEM((CH,), jnp.int32),
        pltpu.VMEM((CH,), jnp.int32),
        pltpu.VMEM((CH // 64, 64), jnp.int32),
        pltpu.VMEM((CH // 64, 64), jnp.int32),
        pltpu.VMEM((QCHUNKS, 128), jnp.int32),
        pltpu.VMEM((QCHUNKS * 128,), jnp.int32),
        pltpu.VMEM((QCHUNKS, 128), jnp.int32),
        pltpu.VMEM((K27 * CH,), jnp.int32),
        pltpu.VMEM((2048,), jnp.int32),
        pltpu.SemaphoreType.DMA,
    ],
)


GCHUNK = 120
NGCH = (K27 * CH) // GCHUNK


def _gather_body(xpad, fidx, g_out, xsh, idx_v, rows_a, rows_b, sem_g, sem_w):
    c = lax.axis_index("c")
    s = lax.axis_index("s")
    wid = c * NS + s
    gbase = wid * K27 * CH

    cps = XROWS // NS
    off = 0
    for ln in (120, 120, 120, 120, 120, 48):
        r0 = s * cps + off
        pltpu.sync_copy(xpad.at[pl.ds(r0, ln)], rows_a.at[pl.ds(0, ln)])
        pltpu.sync_copy(rows_a.at[pl.ds(0, ln)], xsh.at[pl.ds(r0, ln)])
        off += ln
    plsc.subcore_barrier()

    pltpu.sync_copy(fidx.at[pl.ds(gbase, K27 * CH)], idx_v)
    bufs = (rows_a, rows_b)
    gd = [None] * NGCH
    wd = [None] * NGCH
    for ci in range(NGCH):
        if ci >= 2:
            wd[ci - 2].wait()
        gd[ci] = pltpu.async_copy(
            xsh.at[idx_v.at[pl.ds(ci * GCHUNK, GCHUNK)]],
            bufs[ci % 2], sem_g)
        if ci >= 1:
            gd[ci - 1].wait()
            wd[ci - 1] = pltpu.async_copy(
                bufs[(ci - 1) % 2],
                g_out.at[wid, pl.ds((ci - 1) * GCHUNK, GCHUNK)], sem_w)
    gd[NGCH - 1].wait()
    wd[NGCH - 1] = pltpu.async_copy(
        bufs[(NGCH - 1) % 2],
        g_out.at[wid, pl.ds((NGCH - 1) * GCHUNK, GCHUNK)], sem_w)
    wd[NGCH - 2].wait()
    wd[NGCH - 1].wait()


_gather_kernel = pl.kernel(
    _gather_body,
    out_type=jax.ShapeDtypeStruct((NW, K27 * CH, C), jnp.float32),
    mesh=_mesh,
    scratch_types=[
        pltpu.VMEM_SHARED((XROWS, C), jnp.float32),
        pltpu.VMEM((K27 * CH,), jnp.int32),
        pltpu.VMEM((GCHUNK, C), jnp.float32),
        pltpu.VMEM((GCHUNK, C), jnp.float32),
        pltpu.SemaphoreType.DMA,
        pltpu.SemaphoreType.DMA,
    ],
)


def _bn_relu_body(x_ref, w_ref, b_ref, o_ref):
    x = x_ref[...]
    ri = lax.broadcasted_iota(jnp.int32, (NPAD, 1), 0)
    msk = ri < N
    mean = jnp.sum(x, axis=0, keepdims=True) * (1.0 / N)
    d = x - mean
    dm = jnp.where(msk, d, 0.0)
    var = jnp.sum(dm * dm, axis=0, keepdims=True) * (1.0 / N)
    y = d * lax.rsqrt(var + EPS) * w_ref[...] + b_ref[...]
    y = jnp.maximum(y, 0.0)
    o_ref[pl.ds(0, NPAD), :] = jnp.where(msk, y, 0.0)
    o_ref[pl.ds(NPAD, XROWS - NPAD), :] = jnp.zeros((XROWS - NPAD, C),
                                                    jnp.float32)


def _bn_relu(xpad, w, b):
    return pl.pallas_call(
        _bn_relu_body,
        out_shape=jax.ShapeDtypeStruct((XROWS, C), jnp.float32),
    )(xpad, w.reshape(1, C), b.reshape(1, C))


BW = 32
MR = BW * RB


def _mm_body(g_ref, w_ref, o_ref):
    k = pl.program_id(1)
    gm = g_ref[...].reshape(MR, C)
    part = jnp.dot(gm, w_ref[0], preferred_element_type=jnp.float32)

    @pl.when(k == 0)
    def _():
        o_ref[...] = part

    @pl.when(k > 0)
    def _():
        o_ref[...] += part


def _mm_res_body(g_ref, w_ref, f_ref, o_ref):
    k = pl.program_id(1)
    gm = g_ref[...].reshape(MR, C)
    part = jnp.dot(gm, w_ref[0], preferred_element_type=jnp.float32)

    @pl.when(k == 0)
    def _():
        o_ref[...] = f_ref[...] + part

    @pl.when(k > 0)
    def _():
        o_ref[...] += part


def _conv_mm(g, w):
    return pl.pallas_call(
        _mm_body,
        grid=(NPAD // MR, K27),
        in_specs=[
            pl.BlockSpec((BW, RB, C), lambda j, k: (j, k, 0)),
            pl.BlockSpec((1, C, C), lambda j, k: (k, 0, 0)),
        ],
        out_specs=pl.BlockSpec((MR, C), lambda j, k: (j, 0)),
        out_shape=jax.ShapeDtypeStruct((NPAD, C), jnp.float32),
    )(g, w)


def _conv_mm_res(g, w, fpad):
    return pl.pallas_call(
        _mm_res_body,
        grid=(NPAD // MR, K27),
        in_specs=[
            pl.BlockSpec((BW, RB, C), lambda j, k: (j, k, 0)),
            pl.BlockSpec((1, C, C), lambda j, k: (k, 0, 0)),
            pl.BlockSpec((MR, C), lambda j, k: (j, 0)),
        ],
        out_specs=pl.BlockSpec((MR, C), lambda j, k: (j, 0)),
        out_shape=jax.ShapeDtypeStruct((NPAD, C), jnp.float32),
    )(g, w, fpad)


def kernel(feat_list, pos_list, W1, W2, bn1_w, bn1_b, bn2_w, bn2_b):
    pos_i = pos_list.astype(jnp.int32)
    j = jnp.arange(NPAD - N, dtype=jnp.int32)
    pad_pos = jnp.stack([jnp.full_like(j, G), j // G, j % G], axis=1)
    posT = jnp.concatenate([pos_i, pad_pos], axis=0).T
    posx, posy, posz = posT[0], posT[1], posT[2]

    fidx = _nbr_kernel(posx, posy, posz)

    featpad = jnp.pad(feat_list, ((0, NPAD - N), (0, 0)))
    x1 = _bn_relu(featpad, bn1_w, bn1_b)
    g1 = _gather_kernel(x1, fidx)
    h = _conv_mm(g1, W1)
    x2 = _bn_relu(h, bn2_w, bn2_b)
    g2 = _gather_kernel(x2, fidx)
    outpad = _conv_mm_res(g2, W2, featpad)
    return outpad[:N]
